# baseline (speedup 1.0000x reference)
import functools

import jax
import jax.numpy as jnp
from jax import lax
from jax.experimental import pallas as pl
from jax.experimental.pallas import tpu as pltpu
from jax.experimental.pallas import tpu_sc as plsc

NC, NS = 2, 16
NW = NC * NS
CHUNK = 128
SLOTS = 2
DEG_W = 16


def _mm_body(x_ref, w_ref, o_ref):
    o_ref[...] = jnp.dot(x_ref[...], w_ref[...],
                         preferred_element_type=jnp.float32)


def _scale_body(h_ref, d_ref, hp_ref, dinv_ref):
    deg = d_ref[0][:, :1] + d_ref[1][:, :1] + 1.0
    dinv = lax.rsqrt(deg)
    dinv_ref[...] = dinv
    hp_ref[...] = h_ref[...] * dinv


def _final_body(a_ref, hp_ref, dinv_ref, b_ref, o_ref):
    tot = a_ref[0] + a_ref[1] + hp_ref[...]
    o_ref[...] = tot * dinv_ref[...] + b_ref[...]


def _deg_body(nchunk, epw, rpt, dst_hbm, ones_hbm, zero_hbm, out_hbm,
              onesb, dstb, sem_s, degsh):
    c = lax.axis_index("c")
    s = lax.axis_index("s")
    wid = c * NS + s
    pltpu.sync_copy(zero_hbm, degsh.at[pl.ds(s * rpt, rpt)])
    pltpu.sync_copy(ones_hbm, onesb)
    plsc.subcore_barrier()
    base = wid * epw

    nit = nchunk // SLOTS

    @pl.loop(0, nit)
    def _(m):
        off = base + m * (SLOTS * CHUNK)
        p = m & 1
        for j in range(SLOTS):
            pltpu.sync_copy(dst_hbm.at[pl.ds(off + j * CHUNK, CHUNK)],
                            dstb.at[p, j])

        @pl.when(m > 0)
        def _():
            for j in range(SLOTS):
                pltpu.make_async_copy(ones_hbm, onesb, sem_s).wait()

        for j in range(SLOTS):
            pltpu.async_copy(onesb, degsh.at[dstb.at[p, j]], sem_s, add=True)

    for j in range(SLOTS):
        pltpu.make_async_copy(ones_hbm, onesb, sem_s).wait()
    plsc.subcore_barrier()
    pltpu.sync_copy(degsh.at[pl.ds(s * rpt, rpt)],
                    out_hbm.at[c, pl.ds(s * rpt, rpt)])


def _scat_body(nchunk, epw, rpt, n, hp_hbm, src_hbm, dst_hbm, zero_hbm,
               out_hbm, srcb, dstb, rows, sem_g, sem_s, accsh, hpsh):
    c = lax.axis_index("c")
    s = lax.axis_index("s")
    wid = c * NS + s
    pltpu.sync_copy(zero_hbm, accsh.at[pl.ds(s * rpt, rpt)])
    spr = (n // (8 * NS)) * 8
    tail = n - spr * NS
    pltpu.sync_copy(hp_hbm.at[pl.ds(s * spr, spr)],
                    hpsh.at[pl.ds(s * spr, spr)])

    @pl.when(jnp.logical_and(s == 0, tail > 0))
    def _():
        pltpu.sync_copy(hp_hbm.at[pl.ds(spr * NS, tail)],
                        hpsh.at[pl.ds(spr * NS, tail)])

    plsc.subcore_barrier()
    base = wid * epw
    nit = nchunk // SLOTS

    @pl.loop(0, nit)
    def _(m):
        off = base + m * (SLOTS * CHUNK)
        p = m & 1
        for j in range(SLOTS):
            pltpu.sync_copy(src_hbm.at[pl.ds(off + j * CHUNK, CHUNK)],
                            srcb.at[j])
            pltpu.sync_copy(dst_hbm.at[pl.ds(off + j * CHUNK, CHUNK)],
                            dstb.at[p, j])
        gs = [pltpu.async_copy(hpsh.at[srcb.at[j]], rows.at[p, j], sem_g)
              for j in range(SLOTS)]

        @pl.when(m > 0)
        def _():
            for j in range(SLOTS):
                pltpu.make_async_copy(hp_hbm.at[pl.ds(0, CHUNK)],
                                      rows.at[0, 0], sem_s).wait()

        for j in range(SLOTS):
            gs[j].wait()
            pltpu.async_copy(rows.at[p, j], accsh.at[dstb.at[p, j]], sem_s,
                             add=True)

    for j in range(SLOTS):
        pltpu.make_async_copy(hp_hbm.at[pl.ds(0, CHUNK)],
                              rows.at[0, 0], sem_s).wait()
    plsc.subcore_barrier()
    pltpu.sync_copy(accsh.at[pl.ds(s * rpt, rpt)],
                    out_hbm.at[c, pl.ds(s * rpt, rpt)])


def kernel(x, edge_index, W_mu, b_mu, W_log, b_log):
    n, k = x.shape
    lat = W_mu.shape[1]
    c2 = 2 * lat
    e = edge_index.shape[1]

    rpt = -(-(-(-(n + 1) // NS)) // 8) * 8
    np_ = rpt * NS
    nchunk = -(-e // (NW * CHUNK * SLOTS)) * SLOTS
    e_pad = nchunk * NW * CHUNK
    epw = nchunk * CHUNK

    src = edge_index[0].astype(jnp.int32)
    dst = edge_index[1].astype(jnp.int32)
    pad = e_pad - e
    src_p = jnp.concatenate([src, jnp.zeros((pad,), jnp.int32)])
    dst_p = jnp.concatenate([dst, jnp.full((pad,), n, jnp.int32)])

    w_cat = jnp.concatenate([W_mu, W_log], axis=1)
    b_cat = jnp.concatenate([b_mu, b_log])[None, :]

    ones_rows = jnp.ones((CHUNK, DEG_W), jnp.float32)
    zdeg = jnp.zeros((rpt, DEG_W), jnp.float32)
    zacc = jnp.zeros((rpt, c2), jnp.float32)

    bm = 400
    h = pl.pallas_call(
        _mm_body,
        grid=(n // bm,),
        in_specs=[pl.BlockSpec((bm, k), lambda i: (i, 0)),
                  pl.BlockSpec((k, c2), lambda i: (0, 0))],
        out_specs=pl.BlockSpec((bm, c2), lambda i: (i, 0)),
        out_shape=jax.ShapeDtypeStruct((n, c2), jnp.float32),
    )(x, w_cat)

    mesh = plsc.VectorSubcoreMesh(core_axis_name="c", subcore_axis_name="s")
    degp = pl.kernel(
        functools.partial(_deg_body, nchunk, epw, rpt),
        out_type=jax.ShapeDtypeStruct((NC, np_, DEG_W), jnp.float32),
        mesh=mesh,
        compiler_params=pltpu.CompilerParams(use_tc_tiling_on_sc=False),
        scratch_types=[
            pltpu.VMEM((CHUNK, DEG_W), jnp.float32),
            pltpu.VMEM((2, SLOTS, CHUNK), jnp.int32),
            pltpu.SemaphoreType.DMA,
            pltpu.VMEM_SHARED((np_, DEG_W), jnp.float32),
        ],
    )(dst_p, ones_rows, zdeg)

    bs = 400
    hp, dinv = pl.pallas_call(
        _scale_body,
        grid=(n // bs,),
        in_specs=[pl.BlockSpec((bs, c2), lambda i: (i, 0)),
                  pl.BlockSpec((NC, bs, DEG_W), lambda i: (0, i, 0))],
        out_specs=[pl.BlockSpec((bs, c2), lambda i: (i, 0)),
                   pl.BlockSpec((bs, 1), lambda i: (i, 0))],
        out_shape=[jax.ShapeDtypeStruct((n, c2), jnp.float32),
                   jax.ShapeDtypeStruct((n, 1), jnp.float32)],
    )(h, degp)

    accp = pl.kernel(
        functools.partial(_scat_body, nchunk, epw, rpt, n),
        out_type=jax.ShapeDtypeStruct((NC, np_, c2), jnp.float32),
        mesh=mesh,
        compiler_params=pltpu.CompilerParams(use_tc_tiling_on_sc=False),
        scratch_types=[
            pltpu.VMEM((SLOTS, CHUNK), jnp.int32),
            pltpu.VMEM((2, SLOTS, CHUNK), jnp.int32),
            pltpu.VMEM((2, SLOTS, CHUNK, c2), jnp.float32),
            pltpu.SemaphoreType.DMA,
            pltpu.SemaphoreType.DMA,
            pltpu.VMEM_SHARED((np_, c2), jnp.float32),
            pltpu.VMEM_SHARED((n, c2), jnp.float32),
        ],
    )(hp, src_p, dst_p, zacc)

    out = pl.pallas_call(
        _final_body,
        grid=(n // bs,),
        in_specs=[pl.BlockSpec((NC, bs, c2), lambda i: (0, i, 0)),
                  pl.BlockSpec((bs, c2), lambda i: (i, 0)),
                  pl.BlockSpec((bs, 1), lambda i: (i, 0)),
                  pl.BlockSpec((1, c2), lambda i: (0, 0))],
        out_specs=pl.BlockSpec((bs, c2), lambda i: (i, 0)),
        out_shape=jax.ShapeDtypeStruct((n, c2), jnp.float32),
    )(accp, hp, dinv, b_cat)

    return out[:, :lat], out[:, lat:]


# speedup vs baseline: 1.0991x; 1.0991x over previous
import functools

import jax
import jax.numpy as jnp
from jax import lax
from jax.experimental import pallas as pl
from jax.experimental.pallas import tpu as pltpu
from jax.experimental.pallas import tpu_sc as plsc

NC, NS = 2, 16
NW = NC * NS
CHUNK = 512
DEG_W = 16


def _mm_body(x_ref, w_ref, d_ref, hp_ref, dinv_ref):
    deg = d_ref[0][:, :1] + d_ref[1][:, :1] + 1.0
    dinv = lax.rsqrt(deg)
    dinv_ref[...] = dinv
    h = jnp.dot(x_ref[...], w_ref[...], preferred_element_type=jnp.float32)
    hp_ref[...] = h * dinv


def _final_body(a_ref, hp_ref, dinv_ref, b_ref, o_ref):
    tot = a_ref[0] + a_ref[1] + hp_ref[...]
    o_ref[...] = tot * dinv_ref[...] + b_ref[...]


def _deg_body(nchunk, epw, rpt, dst_hbm, ones_hbm, zero_hbm, out_hbm,
              onesb, dstb, sem_s, degsh):
    c = lax.axis_index("c")
    s = lax.axis_index("s")
    wid = c * NS + s
    pltpu.sync_copy(zero_hbm, degsh.at[pl.ds(s * rpt, rpt)])
    pltpu.sync_copy(ones_hbm, onesb)
    plsc.subcore_barrier()
    base = wid * epw

    @pl.loop(0, nchunk)
    def _(i):
        pltpu.sync_copy(dst_hbm.at[pl.ds(base + i * CHUNK, CHUNK)], dstb)
        pltpu.async_copy(onesb, degsh.at[dstb], sem_s, add=True).wait()

    plsc.subcore_barrier()
    pltpu.sync_copy(degsh.at[pl.ds(s * rpt, rpt)],
                    out_hbm.at[c, pl.ds(s * rpt, rpt)])


def _scat_body(nchunk, epw, rpt, n, hp_hbm, src_hbm, dst_hbm, zero_hbm,
               out_hbm, srcb, dstb, rows, sem_g, sem_s, accsh, hpsh):
    c = lax.axis_index("c")
    s = lax.axis_index("s")
    wid = c * NS + s
    pltpu.sync_copy(zero_hbm, accsh.at[pl.ds(s * rpt, rpt)])
    spr = (n // (8 * NS)) * 8
    tail = n - spr * NS
    pltpu.sync_copy(hp_hbm.at[pl.ds(s * spr, spr)],
                    hpsh.at[pl.ds(s * spr, spr)])

    @pl.when(jnp.logical_and(s == 0, tail > 0))
    def _():
        pltpu.sync_copy(hp_hbm.at[pl.ds(spr * NS, tail)],
                        hpsh.at[pl.ds(spr * NS, tail)])

    plsc.subcore_barrier()
    base = wid * epw

    @pl.loop(0, nchunk)
    def _(i):
        off = base + i * CHUNK
        pltpu.sync_copy(src_hbm.at[pl.ds(off, CHUNK)], srcb)
        pltpu.sync_copy(dst_hbm.at[pl.ds(off, CHUNK)], dstb)
        pltpu.async_copy(hpsh.at[srcb], rows, sem_g).wait()
        pltpu.async_copy(rows, accsh.at[dstb], sem_s, add=True).wait()

    plsc.subcore_barrier()
    pltpu.sync_copy(accsh.at[pl.ds(s * rpt, rpt)],
                    out_hbm.at[c, pl.ds(s * rpt, rpt)])


def kernel(x, edge_index, W_mu, b_mu, W_log, b_log):
    n, k = x.shape
    lat = W_mu.shape[1]
    c2 = 2 * lat
    e = edge_index.shape[1]

    rpt = -(-(-(-(n + 1) // NS)) // 8) * 8
    np_ = rpt * NS
    nchunk = -(-e // (NW * CHUNK))
    e_pad = nchunk * NW * CHUNK
    epw = nchunk * CHUNK

    src = edge_index[0].astype(jnp.int32)
    dst = edge_index[1].astype(jnp.int32)
    pad = e_pad - e
    src_p = jnp.concatenate([src, jnp.zeros((pad,), jnp.int32)])
    dst_p = jnp.concatenate([dst, jnp.full((pad,), n, jnp.int32)])

    w_cat = jnp.concatenate([W_mu, W_log], axis=1)
    b_cat = jnp.concatenate([b_mu, b_log])[None, :]

    ones_rows = jnp.ones((CHUNK, DEG_W), jnp.float32)
    zdeg = jnp.zeros((rpt, DEG_W), jnp.float32)
    zacc = jnp.zeros((rpt, c2), jnp.float32)

    mesh = plsc.VectorSubcoreMesh(core_axis_name="c", subcore_axis_name="s")
    degp = pl.kernel(
        functools.partial(_deg_body, nchunk, epw, rpt),
        out_type=jax.ShapeDtypeStruct((NC, np_, DEG_W), jnp.float32),
        mesh=mesh,
        compiler_params=pltpu.CompilerParams(use_tc_tiling_on_sc=False),
        scratch_types=[
            pltpu.VMEM((CHUNK, DEG_W), jnp.float32),
            pltpu.VMEM((CHUNK,), jnp.int32),
            pltpu.SemaphoreType.DMA,
            pltpu.VMEM_SHARED((np_, DEG_W), jnp.float32),
        ],
    )(dst_p, ones_rows, zdeg)

    bm = 400
    hp, dinv = pl.pallas_call(
        _mm_body,
        grid=(n // bm,),
        in_specs=[pl.BlockSpec((bm, k), lambda i: (i, 0)),
                  pl.BlockSpec((k, c2), lambda i: (0, 0)),
                  pl.BlockSpec((NC, bm, DEG_W), lambda i: (0, i, 0))],
        out_specs=[pl.BlockSpec((bm, c2), lambda i: (i, 0)),
                   pl.BlockSpec((bm, 1), lambda i: (i, 0))],
        out_shape=[jax.ShapeDtypeStruct((n, c2), jnp.float32),
                   jax.ShapeDtypeStruct((n, 1), jnp.float32)],
    )(x, w_cat, degp)

    accp = pl.kernel(
        functools.partial(_scat_body, nchunk, epw, rpt, n),
        out_type=jax.ShapeDtypeStruct((NC, np_, c2), jnp.float32),
        mesh=mesh,
        compiler_params=pltpu.CompilerParams(use_tc_tiling_on_sc=False),
        scratch_types=[
            pltpu.VMEM((CHUNK,), jnp.int32),
            pltpu.VMEM((CHUNK,), jnp.int32),
            pltpu.VMEM((CHUNK, c2), jnp.float32),
            pltpu.SemaphoreType.DMA,
            pltpu.SemaphoreType.DMA,
            pltpu.VMEM_SHARED((np_, c2), jnp.float32),
            pltpu.VMEM_SHARED((n, c2), jnp.float32),
        ],
    )(hp, src_p, dst_p, zacc)

    bs = 400
    out = pl.pallas_call(
        _final_body,
        grid=(n // bs,),
        in_specs=[pl.BlockSpec((NC, bs, c2), lambda i: (0, i, 0)),
                  pl.BlockSpec((bs, c2), lambda i: (i, 0)),
                  pl.BlockSpec((bs, 1), lambda i: (i, 0)),
                  pl.BlockSpec((1, c2), lambda i: (0, 0))],
        out_specs=pl.BlockSpec((bs, c2), lambda i: (i, 0)),
        out_shape=jax.ShapeDtypeStruct((n, c2), jnp.float32),
    )(accp, hp, dinv, b_cat)

    return out[:, :lat], out[:, lat:]


# speedup vs baseline: 1.1407x; 1.0378x over previous
import functools

import jax
import jax.numpy as jnp
from jax import lax
from jax.experimental import pallas as pl
from jax.experimental.pallas import tpu as pltpu
from jax.experimental.pallas import tpu_sc as plsc

NC, NS = 2, 16
NW = NC * NS
CHUNK = 512
DEG_W = 16


def _mm_body(x_ref, w_ref, o_ref):
    o_ref[...] = jnp.dot(x_ref[...], w_ref[...],
                         preferred_element_type=jnp.float32)


def _scale_body(h_ref, d_ref, hp_ref, dinv_ref):
    deg = d_ref[0][:, :1] + d_ref[1][:, :1] + 1.0
    dinv = lax.rsqrt(deg)
    dinv_ref[...] = dinv
    hp_ref[...] = h_ref[...] * dinv


def _final_body(a_ref, hp_ref, dinv_ref, b_ref, o_ref):
    tot = a_ref[0] + a_ref[1] + hp_ref[...]
    o_ref[...] = tot * dinv_ref[...] + b_ref[...]


def _deg_body(nchunk, epw, rpt, dst_hbm, ones_hbm, zero_hbm, out_hbm,
              onesb, dstb, sem_s, degsh):
    c = lax.axis_index("c")
    s = lax.axis_index("s")
    wid = c * NS + s
    pltpu.sync_copy(zero_hbm, degsh.at[pl.ds(s * rpt, rpt)])
    pltpu.sync_copy(ones_hbm, onesb)
    plsc.subcore_barrier()
    base = wid * epw

    @pl.loop(0, nchunk)
    def _(i):
        pltpu.sync_copy(dst_hbm.at[pl.ds(base + i * CHUNK, CHUNK)], dstb)
        pltpu.async_copy(onesb, degsh.at[dstb], sem_s, add=True).wait()

    plsc.subcore_barrier()
    pltpu.sync_copy(degsh.at[pl.ds(s * rpt, rpt)],
                    out_hbm.at[c, pl.ds(s * rpt, rpt)])


def _scat_body(nchunk, epw, rpt, n, hp_hbm, src_hbm, dst_hbm, zero_hbm,
               out_hbm, srcb, dstb, rows, sem_g, sem_s, accsh, hpsh):
    c = lax.axis_index("c")
    s = lax.axis_index("s")
    wid = c * NS + s
    pltpu.sync_copy(zero_hbm, accsh.at[pl.ds(s * rpt, rpt)])
    spr = (n // (8 * NS)) * 8
    tail = n - spr * NS
    pltpu.sync_copy(hp_hbm.at[pl.ds(s * spr, spr)],
                    hpsh.at[pl.ds(s * spr, spr)])

    @pl.when(jnp.logical_and(s == 0, tail > 0))
    def _():
        pltpu.sync_copy(hp_hbm.at[pl.ds(spr * NS, tail)],
                        hpsh.at[pl.ds(spr * NS, tail)])

    plsc.subcore_barrier()
    base = wid * epw

    @pl.loop(0, nchunk)
    def _(i):
        off = base + i * CHUNK
        pltpu.sync_copy(src_hbm.at[pl.ds(off, CHUNK)], srcb)
        pltpu.sync_copy(dst_hbm.at[pl.ds(off, CHUNK)], dstb)
        pltpu.async_copy(hpsh.at[srcb], rows, sem_g).wait()
        pltpu.async_copy(rows, accsh.at[dstb], sem_s, add=True).wait()

    plsc.subcore_barrier()
    pltpu.sync_copy(accsh.at[pl.ds(s * rpt, rpt)],
                    out_hbm.at[c, pl.ds(s * rpt, rpt)])


def kernel(x, edge_index, W_mu, b_mu, W_log, b_log):
    n, k = x.shape
    lat = W_mu.shape[1]
    c2 = 2 * lat
    e = edge_index.shape[1]

    rpt = -(-(-(-(n + 1) // NS)) // 8) * 8
    np_ = rpt * NS
    nchunk = -(-e // (NW * CHUNK))
    e_pad = nchunk * NW * CHUNK
    epw = nchunk * CHUNK

    src = edge_index[0].astype(jnp.int32)
    dst = edge_index[1].astype(jnp.int32)
    pad = e_pad - e
    src_p = jnp.concatenate([src, jnp.zeros((pad,), jnp.int32)])
    dst_p = jnp.concatenate([dst, jnp.full((pad,), n, jnp.int32)])

    w_cat = jnp.concatenate([W_mu, W_log], axis=1)
    b_cat = jnp.concatenate([b_mu, b_log])[None, :]

    ones_rows = jnp.ones((CHUNK, DEG_W), jnp.float32)
    zdeg = jnp.zeros((rpt, DEG_W), jnp.float32)
    zacc = jnp.zeros((rpt, c2), jnp.float32)

    mesh = plsc.VectorSubcoreMesh(core_axis_name="c", subcore_axis_name="s")
    degp = pl.kernel(
        functools.partial(_deg_body, nchunk, epw, rpt),
        out_type=jax.ShapeDtypeStruct((NC, np_, DEG_W), jnp.float32),
        mesh=mesh,
        compiler_params=pltpu.CompilerParams(use_tc_tiling_on_sc=False),
        scratch_types=[
            pltpu.VMEM((CHUNK, DEG_W), jnp.float32),
            pltpu.VMEM((CHUNK,), jnp.int32),
            pltpu.SemaphoreType.DMA,
            pltpu.VMEM_SHARED((np_, DEG_W), jnp.float32),
        ],
    )(dst_p, ones_rows, zdeg)

    bm = 400
    h = pl.pallas_call(
        _mm_body,
        grid=(n // bm,),
        in_specs=[pl.BlockSpec((bm, k), lambda i: (i, 0)),
                  pl.BlockSpec((k, c2), lambda i: (0, 0))],
        out_specs=pl.BlockSpec((bm, c2), lambda i: (i, 0)),
        out_shape=jax.ShapeDtypeStruct((n, c2), jnp.float32),
    )(x, w_cat)

    hp, dinv = pl.pallas_call(
        _scale_body,
        grid=(n // bm,),
        in_specs=[pl.BlockSpec((bm, c2), lambda i: (i, 0)),
                  pl.BlockSpec((NC, bm, DEG_W), lambda i: (0, i, 0))],
        out_specs=[pl.BlockSpec((bm, c2), lambda i: (i, 0)),
                   pl.BlockSpec((bm, 1), lambda i: (i, 0))],
        out_shape=[jax.ShapeDtypeStruct((n, c2), jnp.float32),
                   jax.ShapeDtypeStruct((n, 1), jnp.float32)],
    )(h, degp)

    accp = pl.kernel(
        functools.partial(_scat_body, nchunk, epw, rpt, n),
        out_type=jax.ShapeDtypeStruct((NC, np_, c2), jnp.float32),
        mesh=mesh,
        compiler_params=pltpu.CompilerParams(use_tc_tiling_on_sc=False),
        scratch_types=[
            pltpu.VMEM((CHUNK,), jnp.int32),
            pltpu.VMEM((CHUNK,), jnp.int32),
            pltpu.VMEM((CHUNK, c2), jnp.float32),
            pltpu.SemaphoreType.DMA,
            pltpu.SemaphoreType.DMA,
            pltpu.VMEM_SHARED((np_, c2), jnp.float32),
            pltpu.VMEM_SHARED((n, c2), jnp.float32),
        ],
    )(hp, src_p, dst_p, zacc)

    bs = 400
    out = pl.pallas_call(
        _final_body,
        grid=(n // bs,),
        in_specs=[pl.BlockSpec((NC, bs, c2), lambda i: (0, i, 0)),
                  pl.BlockSpec((bs, c2), lambda i: (i, 0)),
                  pl.BlockSpec((bs, 1), lambda i: (i, 0)),
                  pl.BlockSpec((1, c2), lambda i: (0, 0))],
        out_specs=pl.BlockSpec((bs, c2), lambda i: (i, 0)),
        out_shape=jax.ShapeDtypeStruct((n, c2), jnp.float32),
    )(accp, hp, dinv, b_cat)

    return out[:, :lat], out[:, lat:]


# speedup vs baseline: 1.1594x; 1.0164x over previous
import functools

import jax
import jax.numpy as jnp
from jax import lax
from jax.experimental import pallas as pl
from jax.experimental.pallas import tpu as pltpu
from jax.experimental.pallas import tpu_sc as plsc

NC, NS = 2, 16
NW = NC * NS
CHUNK = 704
DEG_W = 16


def _mm_body(x_ref, w_ref, o_ref):
    o_ref[...] = jnp.dot(x_ref[...], w_ref[...],
                         preferred_element_type=jnp.float32)


def _scale_body(h_ref, d_ref, hp_ref, dinv_ref):
    deg = d_ref[0][:, :1] + d_ref[1][:, :1] + 1.0
    dinv = lax.rsqrt(deg)
    dinv_ref[...] = dinv
    hp_ref[...] = h_ref[...] * dinv


def _final_body(a_ref, hp_ref, dinv_ref, b_ref, o_ref):
    tot = a_ref[0] + a_ref[1] + hp_ref[...]
    o_ref[...] = tot * dinv_ref[...] + b_ref[...]


def _deg_body(nchunk, epw, rpt, dst_hbm, ones_hbm, zero_hbm, out_hbm,
              onesb, dstb, sem_s, degsh):
    c = lax.axis_index("c")
    s = lax.axis_index("s")
    wid = c * NS + s
    pltpu.sync_copy(zero_hbm, degsh.at[pl.ds(s * rpt, rpt)])
    pltpu.sync_copy(ones_hbm, onesb)
    plsc.subcore_barrier()
    base = wid * epw

    @pl.loop(0, nchunk)
    def _(i):
        pltpu.sync_copy(dst_hbm.at[pl.ds(base + i * CHUNK, CHUNK)], dstb)
        pltpu.async_copy(onesb, degsh.at[dstb], sem_s, add=True).wait()

    plsc.subcore_barrier()
    pltpu.sync_copy(degsh.at[pl.ds(s * rpt, rpt)],
                    out_hbm.at[c, pl.ds(s * rpt, rpt)])


def _scat_body(nchunk, epw, rpt, n, hp_hbm, src_hbm, dst_hbm, zero_hbm,
               out_hbm, srcb, dstb, rows, sem_g, sem_s, accsh, hpsh):
    c = lax.axis_index("c")
    s = lax.axis_index("s")
    wid = c * NS + s
    pltpu.sync_copy(zero_hbm, accsh.at[pl.ds(s * rpt, rpt)])
    spr = (n // (8 * NS)) * 8
    tail = n - spr * NS
    pltpu.sync_copy(hp_hbm.at[pl.ds(s * spr, spr)],
                    hpsh.at[pl.ds(s * spr, spr)])

    @pl.when(jnp.logical_and(s == 0, tail > 0))
    def _():
        pltpu.sync_copy(hp_hbm.at[pl.ds(spr * NS, tail)],
                        hpsh.at[pl.ds(spr * NS, tail)])

    plsc.subcore_barrier()
    base = wid * epw

    @pl.loop(0, nchunk)
    def _(i):
        off = base + i * CHUNK
        pltpu.sync_copy(src_hbm.at[pl.ds(off, CHUNK)], srcb)
        pltpu.sync_copy(dst_hbm.at[pl.ds(off, CHUNK)], dstb)
        pltpu.async_copy(hpsh.at[srcb], rows, sem_g).wait()
        pltpu.async_copy(rows, accsh.at[dstb], sem_s, add=True).wait()

    plsc.subcore_barrier()
    pltpu.sync_copy(accsh.at[pl.ds(s * rpt, rpt)],
                    out_hbm.at[c, pl.ds(s * rpt, rpt)])


def kernel(x, edge_index, W_mu, b_mu, W_log, b_log):
    n, k = x.shape
    lat = W_mu.shape[1]
    c2 = 2 * lat
    e = edge_index.shape[1]

    rpt = -(-(-(-(n + 1) // NS)) // 8) * 8
    np_ = rpt * NS
    nchunk = -(-e // (NW * CHUNK))
    e_pad = nchunk * NW * CHUNK
    epw = nchunk * CHUNK

    src = edge_index[0].astype(jnp.int32)
    dst = edge_index[1].astype(jnp.int32)
    pad = e_pad - e
    src_p = jnp.concatenate([src, jnp.zeros((pad,), jnp.int32)])
    dst_p = jnp.concatenate([dst, jnp.full((pad,), n, jnp.int32)])

    w_cat = jnp.concatenate([W_mu, W_log], axis=1)
    b_cat = jnp.concatenate([b_mu, b_log])[None, :]

    ones_rows = jnp.ones((CHUNK, DEG_W), jnp.float32)
    zdeg = jnp.zeros((rpt, DEG_W), jnp.float32)
    zacc = jnp.zeros((rpt, c2), jnp.float32)

    mesh = plsc.VectorSubcoreMesh(core_axis_name="c", subcore_axis_name="s")
    degp = pl.kernel(
        functools.partial(_deg_body, nchunk, epw, rpt),
        out_type=jax.ShapeDtypeStruct((NC, np_, DEG_W), jnp.float32),
        mesh=mesh,
        compiler_params=pltpu.CompilerParams(use_tc_tiling_on_sc=False),
        scratch_types=[
            pltpu.VMEM((CHUNK, DEG_W), jnp.float32),
            pltpu.VMEM((CHUNK,), jnp.int32),
            pltpu.SemaphoreType.DMA,
            pltpu.VMEM_SHARED((np_, DEG_W), jnp.float32),
        ],
    )(dst_p, ones_rows, zdeg)

    bm = 400
    h = pl.pallas_call(
        _mm_body,
        grid=(n // bm,),
        in_specs=[pl.BlockSpec((bm, k), lambda i: (i, 0)),
                  pl.BlockSpec((k, c2), lambda i: (0, 0))],
        out_specs=pl.BlockSpec((bm, c2), lambda i: (i, 0)),
        out_shape=jax.ShapeDtypeStruct((n, c2), jnp.float32),
    )(x, w_cat)

    hp, dinv = pl.pallas_call(
        _scale_body,
        grid=(n // bm,),
        in_specs=[pl.BlockSpec((bm, c2), lambda i: (i, 0)),
                  pl.BlockSpec((NC, bm, DEG_W), lambda i: (0, i, 0))],
        out_specs=[pl.BlockSpec((bm, c2), lambda i: (i, 0)),
                   pl.BlockSpec((bm, 1), lambda i: (i, 0))],
        out_shape=[jax.ShapeDtypeStruct((n, c2), jnp.float32),
                   jax.ShapeDtypeStruct((n, 1), jnp.float32)],
    )(h, degp)

    accp = pl.kernel(
        functools.partial(_scat_body, nchunk, epw, rpt, n),
        out_type=jax.ShapeDtypeStruct((NC, np_, c2), jnp.float32),
        mesh=mesh,
        compiler_params=pltpu.CompilerParams(use_tc_tiling_on_sc=False),
        scratch_types=[
            pltpu.VMEM((CHUNK,), jnp.int32),
            pltpu.VMEM((CHUNK,), jnp.int32),
            pltpu.VMEM((CHUNK, c2), jnp.float32),
            pltpu.SemaphoreType.DMA,
            pltpu.SemaphoreType.DMA,
            pltpu.VMEM_SHARED((np_, c2), jnp.float32),
            pltpu.VMEM_SHARED((n, c2), jnp.float32),
        ],
    )(hp, src_p, dst_p, zacc)

    bs = 400
    out = pl.pallas_call(
        _final_body,
        grid=(n // bs,),
        in_specs=[pl.BlockSpec((NC, bs, c2), lambda i: (0, i, 0)),
                  pl.BlockSpec((bs, c2), lambda i: (i, 0)),
                  pl.BlockSpec((bs, 1), lambda i: (i, 0)),
                  pl.BlockSpec((1, c2), lambda i: (0, 0))],
        out_specs=pl.BlockSpec((bs, c2), lambda i: (i, 0)),
        out_shape=jax.ShapeDtypeStruct((n, c2), jnp.float32),
    )(accp, hp, dinv, b_cat)

    return out[:, :lat], out[:, lat:]
